# baseline (speedup 1.0000x reference)
import functools

import jax
import jax.numpy as jnp
from jax import lax
from jax.experimental import pallas as pl
from jax.experimental.pallas import tpu as pltpu
from jax.experimental.pallas import tpu_sc as plsc

_H = 64
_V = 64
_K = 6


def _make_sc_hist(B, L):
    info = plsc.get_sparse_core_info()
    NC, NS, NL = info.num_cores, info.num_subcores, info.num_lanes
    NW = NC * NS
    assert B % NW == 0 and L % NL == 0
    rows_per_w = B // NW
    groups = L // NL

    mesh = plsc.VectorSubcoreMesh(core_axis_name="c", subcore_axis_name="s")

    NB = 4
    LS = _V + 1

    @functools.partial(
        pl.kernel,
        out_type=jax.ShapeDtypeStruct((B, _V), jnp.int32),
        mesh=mesh,
        compiler_params=pltpu.CompilerParams(needs_layout_passes=False),
        scratch_types=[
            pltpu.VMEM((rows_per_w, L), jnp.int32),
        ] + [pltpu.VMEM((NL * LS,), jnp.int32) for _ in range(NB)]
          + [pltpu.VMEM((rows_per_w, _V), jnp.int32)],
    )
    def sc_hist(seq_hbm, out_hbm, seq_v, *rest):
        hist_bufs, rows_v = rest[:NB], rest[NB]
        wid = lax.axis_index("s") * NC + lax.axis_index("c")
        base = wid * rows_per_w
        pltpu.sync_copy(seq_hbm.at[pl.ds(base, rows_per_w)], seq_v)

        lanebase = lax.iota(jnp.int32, NL) * LS
        ones = jnp.ones((NL,), jnp.int32)
        zeros = jnp.zeros((NL,), jnp.int32)

        for hbuf in hist_bufs:
            @plsc.parallel_loop(0, NL * LS // NL, 1, unroll=8)
            def zero_body(i, hbuf=hbuf):
                hbuf[pl.ds(i * NL, NL)] = zeros

        prev = [zeros] * (_V // NL)
        for r in range(rows_per_w):

            @plsc.parallel_loop(0, groups, NB, unroll=4)
            def grp_body(g, r=r):
                for u, hbuf in enumerate(hist_bufs):
                    v = seq_v[r, pl.ds((g + u) * NL, NL)]
                    plsc.addupdate_scatter(hbuf, [lanebase + v], ones)

            for j in range(_V // NL):
                acc = zeros
                for hbuf in hist_bufs:
                    for lane_i in range(NL):
                        acc = acc + hbuf[pl.ds(lane_i * LS + j * NL, NL)]
                rows_v[r, pl.ds(j * NL, NL)] = acc - prev[j]
                prev[j] = acc

        pltpu.sync_copy(rows_v, out_hbm.at[pl.ds(base, rows_per_w)])

    return sc_hist


def _dense_body(counts_ref, tail_ref, embed_ref, W1_ref, b1_ref, W2_ref,
                b2_ref, gamma_ref, beta_ref, Wq_ref, bq_ref, Wout_ref,
                bout_ref, out_ref):
    f32 = jnp.float32
    mm = lambda a, b: lax.dot_general(a, b, (((1,), (0,)), ((), ())),
                                      preferred_element_type=f32)
    e = embed_ref[...]
    a1 = jnp.maximum(mm(e, W1_ref[...]) + b1_ref[...], 0.0)
    ff = mm(a1, W2_ref[...]) + b2_ref[...]
    x = e + ff
    mu = jnp.mean(x, axis=-1, keepdims=True)
    var = jnp.mean((x - mu) ** 2, axis=-1, keepdims=True)
    h = (x - mu) / jnp.sqrt(var + 1e-5) * gamma_ref[...] + beta_ref[...]
    h_t = jnp.transpose(h)

    n2_col = jnp.sum(h * h, axis=-1, keepdims=True)
    n2_row = jnp.sum(h_t * h_t, axis=0, keepdims=True)
    G = (n2_col > n2_row).astype(f32)

    iv = lax.broadcasted_iota(jnp.int32, (1, _V), 1)
    oh0 = (tail_ref[:, 0:1] == iv).astype(f32)
    oh1 = (tail_ref[:, 1:2] == iv).astype(f32)
    oh2 = (tail_ref[:, 2:3] == iv).astype(f32)
    cf = counts_ref[...].astype(f32) - oh0 - oh1 - oh2

    C = mm(cf, G)
    m = jnp.minimum(jnp.maximum(float(_K) - C, 0.0), cf)

    h_last = mm(oh2, h)
    q = mm(h_last, Wq_ref[...]) + bq_ref[...]
    logits = mm(q, h_t) * (1.0 / (_H ** 0.5))

    lm = jnp.where(m > 0.0, logits, -1e30)
    mx = jnp.max(lm, axis=-1, keepdims=True)
    p = m * jnp.exp(lm - mx)
    w = p / jnp.sum(p, axis=-1, keepdims=True)
    ctx = mm(w, h)
    out_ref[...] = mm(ctx, Wout_ref[...]) + bout_ref[...]


def kernel(seq, embed, W1, b1, W2, b2, gamma, beta, Wq, bq, Wout, bout):
    B, L = seq.shape
    counts = _make_sc_hist(B, L)(seq)
    tail = lax.slice(seq, (0, L - 3), (B, L))
    row = lambda v: v.reshape(1, -1)
    out = pl.pallas_call(
        _dense_body,
        out_shape=jax.ShapeDtypeStruct((B, _V), jnp.float32),
    )(counts, tail, embed, W1, row(b1), W2, row(b2), row(gamma), row(beta),
      Wq, row(bq), Wout, row(bout))
    return out


# speedup vs baseline: 1.1697x; 1.1697x over previous
import functools

import jax
import jax.numpy as jnp
from jax import lax
from jax.experimental import pallas as pl
from jax.experimental.pallas import tpu as pltpu
from jax.experimental.pallas import tpu_sc as plsc

_H = 64
_V = 64
_K = 6
_LS = _V + 1


def _make_sc_hist(B, L):
    info = plsc.get_sparse_core_info()
    NC, NS, NL = info.num_cores, info.num_subcores, info.num_lanes
    NW = NC * NS
    assert B % NW == 0 and L % (2 * NL) == 0
    rows_per_w = B // NW
    half = L // 2
    hgroups = half // NL
    hwords = NL * _LS

    mesh = plsc.VectorSubcoreMesh(core_axis_name="c", subcore_axis_name="s")

    @functools.partial(
        pl.kernel,
        out_type=jax.ShapeDtypeStruct((B, hwords), jnp.int32),
        mesh=mesh,
        compiler_params=pltpu.CompilerParams(needs_layout_passes=False),
        scratch_types=[
            pltpu.VMEM((rows_per_w, half), jnp.int32),
            pltpu.VMEM((rows_per_w, half), jnp.int32),
        ] + [pltpu.VMEM((hwords,), jnp.int32) for _ in range(4)]
          + [pltpu.SemaphoreType.DMA, pltpu.SemaphoreType.DMA],
    )
    def sc_hist(seq_hbm, out_hbm, seq_a, seq_b, *rest):
        hists, (sem_a, sem_b) = rest[:4], rest[4:]
        wid = lax.axis_index("s") * NC + lax.axis_index("c")
        base = wid * rows_per_w
        cp_a = pltpu.async_copy(
            seq_hbm.at[pl.ds(base, rows_per_w), pl.ds(0, half)], seq_a, sem_a)
        cp_b = pltpu.async_copy(
            seq_hbm.at[pl.ds(base, rows_per_w), pl.ds(half, half)], seq_b,
            sem_b)

        lanebase = lax.iota(jnp.int32, NL) * _LS
        ones = jnp.ones((NL,), jnp.int32)
        zeros = jnp.zeros((NL,), jnp.int32)

        for hbuf in hists:
            @plsc.parallel_loop(0, hwords // NL, 1, unroll=5)
            def zero_body(i, hbuf=hbuf):
                hbuf[pl.ds(i * NL, NL)] = zeros

        cp_a.wait()

        @plsc.parallel_loop(0, hgroups, 1, unroll=4)
        def grp_a(g):
            for r in range(rows_per_w):
                v = seq_a[r, pl.ds(g * NL, NL)]
                plsc.addupdate_scatter(hists[r], [lanebase + v], ones)

        cp_b.wait()

        @plsc.parallel_loop(0, hgroups, 1, unroll=4)
        def grp_b(g):
            for r in range(rows_per_w):
                v = seq_b[r, pl.ds(g * NL, NL)]
                plsc.addupdate_scatter(hists[r], [lanebase + v], ones)

        for r in range(rows_per_w):
            pltpu.sync_copy(hists[r], out_hbm.at[base + r])

    return sc_hist


def _dense_body(raw_ref, tailblk_ref, embed_ref, W1_ref, b1_ref, W2_ref,
                b2_ref, gamma_ref, beta_ref, Wq_ref, bq_ref, Wout_ref,
                bout_ref, out_ref):
    f32 = jnp.float32
    mm = lambda a, b: lax.dot_general(a, b, (((1,), (0,)), ((), ())),
                                      preferred_element_type=f32)
    e = embed_ref[...]
    a1 = jnp.maximum(mm(e, W1_ref[...]) + b1_ref[...], 0.0)
    ff = mm(a1, W2_ref[...]) + b2_ref[...]
    x = e + ff
    mu = jnp.mean(x, axis=-1, keepdims=True)
    var = jnp.mean((x - mu) ** 2, axis=-1, keepdims=True)
    h = (x - mu) / jnp.sqrt(var + 1e-5) * gamma_ref[...] + beta_ref[...]
    h_t = jnp.transpose(h)

    n2_col = jnp.sum(h * h, axis=-1, keepdims=True)
    n2_row = jnp.sum(h_t * h_t, axis=0, keepdims=True)
    G = (n2_col > n2_row).astype(f32)

    raw = raw_ref[...]
    cnt = raw[:, 0:_V]
    for lane in range(1, 16):
        cnt = cnt + raw[:, lane * _LS:lane * _LS + _V]

    iv = lax.broadcasted_iota(jnp.int32, (1, _V), 1)
    oh0 = (tailblk_ref[:, 125:126] == iv).astype(f32)
    oh1 = (tailblk_ref[:, 126:127] == iv).astype(f32)
    oh2 = (tailblk_ref[:, 127:128] == iv).astype(f32)
    cf = cnt.astype(f32) - oh0 - oh1 - oh2

    C = mm(cf, G)
    m = jnp.minimum(jnp.maximum(float(_K) - C, 0.0), cf)

    h_last = mm(oh2, h)
    q = mm(h_last, Wq_ref[...]) + bq_ref[...]
    logits = mm(q, h_t) * (1.0 / (_H ** 0.5))

    lm = jnp.where(m > 0.0, logits, -1e30)
    mx = jnp.max(lm, axis=-1, keepdims=True)
    p = m * jnp.exp(lm - mx)
    w = p / jnp.sum(p, axis=-1, keepdims=True)
    ctx = mm(w, h)
    out_ref[...] = mm(ctx, Wout_ref[...]) + bout_ref[...]


def kernel(seq, embed, W1, b1, W2, b2, gamma, beta, Wq, bq, Wout, bout):
    B, L = seq.shape
    raw = _make_sc_hist(B, L)(seq)
    row = lambda v: v.reshape(1, -1)
    full = lambda a: pl.BlockSpec(a.shape, lambda i: (0,) * a.ndim)
    tail_spec = pl.BlockSpec((B, 128), lambda i: (0, L // 128 - 1))
    args = (raw, seq, embed, W1, row(b1), W2, row(b2), row(gamma), row(beta),
            Wq, row(bq), Wout, row(bout))
    specs = [full(a) for a in args]
    specs[1] = tail_spec
    out = pl.pallas_call(
        _dense_body,
        out_shape=jax.ShapeDtypeStruct((B, _V), jnp.float32),
        grid=(1,),
        in_specs=specs,
        out_specs=full(jax.ShapeDtypeStruct((B, _V), jnp.float32)),
    )(*args)
    return out


# speedup vs baseline: 1.1858x; 1.0138x over previous
import functools

import jax
import jax.numpy as jnp
from jax import lax
from jax.experimental import pallas as pl
from jax.experimental.pallas import tpu as pltpu
from jax.experimental.pallas import tpu_sc as plsc

_H = 64
_V = 64
_K = 6
_LS = _V + 1


def _make_sc_hist(B, L):
    info = plsc.get_sparse_core_info()
    NC, NS, NL = info.num_cores, info.num_subcores, info.num_lanes
    NW = NC * NS
    assert B % NW == 0 and L % (2 * NL) == 0
    rows_per_w = B // NW
    half = L // 2
    hgroups = half // NL
    hwords = NL * _LS

    mesh = plsc.VectorSubcoreMesh(core_axis_name="c", subcore_axis_name="s")

    @functools.partial(
        pl.kernel,
        out_type=jax.ShapeDtypeStruct((B, hwords), jnp.int32),
        mesh=mesh,
        compiler_params=pltpu.CompilerParams(needs_layout_passes=False),
        scratch_types=[
            pltpu.VMEM((rows_per_w, half), jnp.int32),
            pltpu.VMEM((rows_per_w, half), jnp.int32),
        ] + [pltpu.VMEM((hwords,), jnp.int32) for _ in range(4)]
          + [pltpu.SemaphoreType.DMA, pltpu.SemaphoreType.DMA],
    )
    def sc_hist(seq_hbm, out_hbm, seq_a, seq_b, *rest):
        hists, (sem_a, sem_b) = rest[:4], rest[4:]
        wid = lax.axis_index("s") * NC + lax.axis_index("c")
        base = wid * rows_per_w
        cp_a = pltpu.async_copy(
            seq_hbm.at[pl.ds(base, rows_per_w), pl.ds(0, half)], seq_a, sem_a)
        cp_b = pltpu.async_copy(
            seq_hbm.at[pl.ds(base, rows_per_w), pl.ds(half, half)], seq_b,
            sem_b)

        lanebase = lax.iota(jnp.int32, NL) * _LS
        ones = jnp.ones((NL,), jnp.int32)
        zeros = jnp.zeros((NL,), jnp.int32)

        for hbuf in hists:
            @plsc.parallel_loop(0, hwords // NL, 1, unroll=5)
            def zero_body(i, hbuf=hbuf):
                hbuf[pl.ds(i * NL, NL)] = zeros

        cp_a.wait()

        @plsc.parallel_loop(0, hgroups, 1, unroll=4)
        def grp_a(g):
            for r in range(rows_per_w):
                v = seq_a[r, pl.ds(g * NL, NL)]
                plsc.addupdate_scatter(hists[r], [lanebase + v], ones)

        cp_b.wait()

        @plsc.parallel_loop(0, hgroups, 1, unroll=4)
        def grp_b(g):
            for r in range(rows_per_w):
                v = seq_b[r, pl.ds(g * NL, NL)]
                plsc.addupdate_scatter(hists[r], [lanebase + v], ones)

        for r in range(rows_per_w):
            pltpu.sync_copy(hists[r], out_hbm.at[base + r])

    return sc_hist


def _pre_body(tailblk_ref, embed_ref, W1_ref, b1_ref, W2_ref, b2_ref,
              gamma_ref, beta_ref, Wq_ref, bq_ref, G_ref, logits_ref,
              ohsum_ref, h_ref):
    f32 = jnp.float32
    mm = lambda a, b: lax.dot_general(a, b, (((1,), (0,)), ((), ())),
                                      preferred_element_type=f32)
    e = embed_ref[...]
    a1 = jnp.maximum(mm(e, W1_ref[...]) + b1_ref[...], 0.0)
    ff = mm(a1, W2_ref[...]) + b2_ref[...]
    x = e + ff
    mu = jnp.mean(x, axis=-1, keepdims=True)
    var = jnp.mean((x - mu) ** 2, axis=-1, keepdims=True)
    h = (x - mu) / jnp.sqrt(var + 1e-5) * gamma_ref[...] + beta_ref[...]
    h_t = jnp.transpose(h)

    n2_col = jnp.sum(h * h, axis=-1, keepdims=True)
    n2_row = jnp.sum(h_t * h_t, axis=0, keepdims=True)
    G_ref[...] = (n2_col > n2_row).astype(f32)

    iv = lax.broadcasted_iota(jnp.int32, (1, _V), 1)
    oh0 = (tailblk_ref[:, 125:126] == iv).astype(f32)
    oh1 = (tailblk_ref[:, 126:127] == iv).astype(f32)
    oh2 = (tailblk_ref[:, 127:128] == iv).astype(f32)
    ohsum_ref[...] = oh0 + oh1 + oh2

    h_last = mm(oh2, h)
    q = mm(h_last, Wq_ref[...]) + bq_ref[...]
    logits_ref[...] = mm(q, h_t) * (1.0 / (_H ** 0.5))
    h_ref[...] = h


def _post_body(raw_ref, G_ref, logits_ref, ohsum_ref, h_ref, Wout_ref,
               bout_ref, out_ref):
    f32 = jnp.float32
    mm = lambda a, b: lax.dot_general(a, b, (((1,), (0,)), ((), ())),
                                      preferred_element_type=f32)
    raw = raw_ref[...]
    cnt = raw[:, 0:_V]
    for lane in range(1, 16):
        cnt = cnt + raw[:, lane * _LS:lane * _LS + _V]
    cf = cnt.astype(f32) - ohsum_ref[...]

    C = mm(cf, G_ref[...])
    m = jnp.minimum(jnp.maximum(float(_K) - C, 0.0), cf)

    lm = jnp.where(m > 0.0, logits_ref[...], -1e30)
    mx = jnp.max(lm, axis=-1, keepdims=True)
    p = m * jnp.exp(lm - mx)
    w = p / jnp.sum(p, axis=-1, keepdims=True)
    ctx = mm(w, h_ref[...])
    out_ref[...] = mm(ctx, Wout_ref[...]) + bout_ref[...]


def kernel(seq, embed, W1, b1, W2, b2, gamma, beta, Wq, bq, Wout, bout):
    B, L = seq.shape
    f32 = jnp.float32
    raw = _make_sc_hist(B, L)(seq)
    row = lambda v: v.reshape(1, -1)
    full = lambda a: pl.BlockSpec(a.shape, lambda i: (0,) * a.ndim)
    tail_spec = pl.BlockSpec((B, 128), lambda i: (0, L // 128 - 1))

    pre_args = (seq, embed, W1, row(b1), W2, row(b2), row(gamma), row(beta),
                Wq, row(bq))
    pre_specs = [tail_spec] + [full(a) for a in pre_args[1:]]
    sds = jax.ShapeDtypeStruct
    G, logits, ohsum, h = pl.pallas_call(
        _pre_body,
        out_shape=(sds((_V, _V), f32), sds((B, _V), f32), sds((B, _V), f32),
                   sds((_V, _H), f32)),
        grid=(1,),
        in_specs=pre_specs,
        out_specs=[full(sds((_V, _V), f32)), full(sds((B, _V), f32)),
                   full(sds((B, _V), f32)), full(sds((_V, _H), f32))],
    )(*pre_args)

    post_args = (raw, G, logits, ohsum, h, Wout, row(bout))
    out = pl.pallas_call(
        _post_body,
        out_shape=sds((B, _V), f32),
        grid=(1,),
        in_specs=[full(a) for a in post_args],
        out_specs=full(sds((B, _V), f32)),
    )(*post_args)
    return out
